# baseline (speedup 1.0000x reference)
import functools

import jax
import jax.numpy as jnp
from jax import lax
from jax.experimental import pallas as pl
from jax.experimental.pallas import tpu as pltpu
from jax.experimental.pallas import tpu_sc as plsc

B = 4096
D = 1024
K = 64
N_SAMPLES = 1000

ROWS_SC = 1024
ROWS_TC = B - ROWS_SC
TC_GRID = 2
TC_BLOCK = ROWS_TC // TC_GRID

NWORKER = 32
RPW = ROWS_SC // NWORKER
CHUNKS = D // 16
GROUPS = RPW // 16


def _prologue_kernel(u_ref, cw_ref, mus_ref, lv_ref, wiv_ref, wm2_ref, c_ref):
    u = u_ref[:]
    g = -jnp.log(-jnp.log(u)) + cw_ref[:]
    rowmax = jnp.max(g, axis=1, keepdims=True)
    col = jax.lax.broadcasted_iota(jnp.int32, g.shape, 1)
    idx = jnp.where(g == rowmax, col, K)
    amin = jnp.min(idx, axis=1, keepdims=True)
    firsthot = (col == amin).astype(jnp.float32)
    w = jnp.sum(firsthot, axis=0, keepdims=True) / N_SAMPLES

    lv = lv_ref[:]
    iv = jnp.exp(-lv)
    mus = mus_ref[:]
    dot = functools.partial(jax.lax.dot_general,
                            dimension_numbers=(((1,), (0,)), ((), ())),
                            precision=jax.lax.Precision.HIGHEST,
                            preferred_element_type=jnp.float32)
    wiv_ref[:] = dot(w, iv)
    wm2_ref[:] = 2.0 * dot(w, mus * iv)
    t = jnp.sum(mus * mus * iv + lv, axis=1, keepdims=True)
    c_ref[:] = jnp.full((1, 16), dot(w, t)[0, 0] + D * jnp.log(2.0 * jnp.pi),
                        dtype=jnp.float32)


def _run_prologue(u, cw, mus, lv):
    return pl.pallas_call(
        _prologue_kernel,
        out_shape=[
            jax.ShapeDtypeStruct((1, D), jnp.float32),
            jax.ShapeDtypeStruct((1, D), jnp.float32),
            jax.ShapeDtypeStruct((1, 16), jnp.float32),
        ],
    )(u, cw, mus, lv)


def _tc_kernel(wiv_ref, wm2_ref, c_ref, x_ref, elbo_ref, stc_ref, acc_s):
    i = pl.program_id(0)

    @pl.when(i == 0)
    def _init():
        acc_s[0, 0] = 0.0

    xb = x_ref[:]
    row = jnp.sum(xb * (xb * wiv_ref[:] - wm2_ref[:]), axis=1)
    elbo_ref[:] = (-0.5 * (row + c_ref[0, 0])).reshape(TC_BLOCK, 1)
    acc_s[0, 0] += jnp.sum(row)

    @pl.when(i == TC_GRID - 1)
    def _fin():
        stc_ref[:] = jnp.full((1, 1), acc_s[0, 0], dtype=jnp.float32)


def _run_tc(wiv2d, wm2_2d, cvec, x):
    return pl.pallas_call(
        _tc_kernel,
        grid=(TC_GRID,),
        in_specs=[
            pl.BlockSpec((1, D), lambda i: (0, 0)),
            pl.BlockSpec((1, D), lambda i: (0, 0)),
            pl.BlockSpec((1, 16), lambda i: (0, 0)),
            pl.BlockSpec((TC_BLOCK, D), lambda i: (i, 0)),
        ],
        out_specs=[
            pl.BlockSpec((TC_BLOCK, 1), lambda i: (i, 0)),
            pl.BlockSpec((1, 1), lambda i: (0, 0)),
        ],
        out_shape=[
            jax.ShapeDtypeStruct((ROWS_TC, 1), jnp.float32),
            jax.ShapeDtypeStruct((1, 1), jnp.float32),
        ],
        scratch_shapes=[pltpu.SMEM((1, 1), jnp.float32)],
    )(wiv2d, wm2_2d, cvec, x)


_SC_MESH = plsc.VectorSubcoreMesh(core_axis_name="c", subcore_axis_name="s",
                                  num_cores=2, num_subcores=16)

_GATHER_DNUMS = lax.GatherDimensionNumbers(
    offset_dims=(), collapsed_slice_dims=(0,), start_index_map=(0,))


def _lane_shuffle(vec, idx):
    return lax.gather(vec, idx[:, None], _GATHER_DNUMS, slice_sizes=(1,),
                      mode=lax.GatherScatterMode.PROMISE_IN_BOUNDS)


@functools.partial(
    pl.kernel,
    out_type=[
        jax.ShapeDtypeStruct((ROWS_SC,), jnp.float32),
        jax.ShapeDtypeStruct((NWORKER * 16,), jnp.float32),
    ],
    mesh=_SC_MESH,
    scratch_types=[
        pltpu.VMEM((RPW * D,), jnp.float32),
        pltpu.VMEM((D,), jnp.float32),
        pltpu.VMEM((D,), jnp.float32),
        pltpu.VMEM((16,), jnp.float32),
        pltpu.VMEM((RPW,), jnp.float32),
        pltpu.VMEM((16,), jnp.float32),
    ],
)
def _sc_kernel(x1d_hbm, wiv_hbm, wm2_hbm, c_hbm, elbo_hbm, psum_hbm,
               xblk, wiv_v, wm2_v, c_v, erows, ptmp):
    cidx = lax.axis_index("c")
    sidx = lax.axis_index("s")
    wid = sidx * 2 + cidx
    base = (ROWS_TC + wid * RPW) * D
    pltpu.sync_copy(x1d_hbm.at[pl.ds(base, RPW * D)], xblk)
    pltpu.sync_copy(wiv_hbm, wiv_v)
    pltpu.sync_copy(wm2_hbm, wm2_v)
    pltpu.sync_copy(c_hbm, c_v)
    cs = c_v[...][0]
    lane = lax.broadcasted_iota(jnp.int32, (16,), 0)
    zeros = jnp.zeros((16,), jnp.float32)

    acc_tot = zeros
    for g in range(GROUPS):
        def row_body(r, carry):
            rowvals, acc_tot = carry
            rbase = (g * 16 + r) * D
            acc = zeros
            for ch in range(CHUNKS):
                xv = xblk[pl.ds(rbase + ch * 16, 16)]
                wv = wiv_v[pl.ds(ch * 16, 16)]
                mv = wm2_v[pl.ds(ch * 16, 16)]
                acc = acc + xv * (xv * wv - mv)
            s_vec = acc
            for sh in (8, 4, 2, 1):
                s_vec = s_vec + _lane_shuffle(s_vec,
                                              jnp.bitwise_xor(lane, sh))
            rowvals = jnp.where(lane == r, s_vec, rowvals)
            return rowvals, acc_tot + acc

        rowvals, acc_tot = lax.fori_loop(0, 16, row_body, (zeros, acc_tot))
        erows[pl.ds(g * 16, 16)] = -0.5 * (rowvals + cs)

    ptmp[...] = acc_tot
    pltpu.sync_copy(erows, elbo_hbm.at[pl.ds(wid * RPW, RPW)])
    pltpu.sync_copy(ptmp, psum_hbm.at[pl.ds(wid * 16, 16)])


def kernel(x, categorical_weights, mus, log_var):
    key = jax.random.key(42)
    u = jax.random.uniform(key, (N_SAMPLES, K), jnp.float32,
                           minval=jnp.finfo(jnp.float32).tiny, maxval=1.0)
    cw = categorical_weights.reshape(1, K)

    wiv2d, wm2_2d, cvec = _run_prologue(u, cw, mus, log_var)
    elbo_tc, stc = _run_tc(wiv2d, wm2_2d, cvec, x)
    elbo_sc, psum = _sc_kernel(x.reshape(-1), wiv2d.reshape(-1),
                               wm2_2d.reshape(-1), cvec.reshape(-1))

    elbo = jnp.concatenate([elbo_tc[:, 0], elbo_sc])
    loss = 0.5 * ((stc[0, 0] + jnp.sum(psum)) / B + cvec[0, 0])
    return loss, elbo


# speedup vs baseline: 3.9107x; 3.9107x over previous
import functools

import jax
import jax.numpy as jnp
from jax.experimental import pallas as pl
from jax.experimental.pallas import tpu as pltpu

B = 4096
D = 1024
K = 64
N_SAMPLES = 1000
NSTREAM = 1
GRID = 2
SUB = B // (NSTREAM * GRID)
GROUP = B // NSTREAM


def _mix_kernel(u_ref, cw_ref, mus_ref, lv_ref, *refs):
    x_refs = refs[:NSTREAM]
    elbo_refs = refs[NSTREAM:2 * NSTREAM]
    loss_ref = refs[2 * NSTREAM]
    wiv_s, wmiv_s, c_s, acc_s = refs[2 * NSTREAM + 1:]
    i = pl.program_id(0)

    @pl.when(i == 0)
    def _prologue():
        u = u_ref[:]
        g = -jnp.log(-jnp.log(u)) + cw_ref[:]
        rowmax = jnp.max(g, axis=1, keepdims=True)
        col = jax.lax.broadcasted_iota(jnp.int32, g.shape, 1)
        idx = jnp.where(g == rowmax, col, K)
        amin = jnp.min(idx, axis=1, keepdims=True)
        firsthot = (col == amin).astype(jnp.float32)
        w = jnp.sum(firsthot, axis=0, keepdims=True) / N_SAMPLES

        lv = lv_ref[:]
        iv = jnp.exp(-lv)
        mus = mus_ref[:]
        dot = functools.partial(jax.lax.dot_general,
                                dimension_numbers=(((1,), (0,)), ((), ())),
                                precision=jax.lax.Precision.HIGHEST,
                                preferred_element_type=jnp.float32)
        wiv_s[:] = dot(w, iv)
        wmiv_s[:] = 2.0 * dot(w, mus * iv)
        t = jnp.sum(mus * mus * iv + lv, axis=1, keepdims=True)
        c_s[0, 0] = dot(w, t)[0, 0] + D * jnp.log(2.0 * jnp.pi)
        acc_s[0, 0] = 0.0

    c = c_s[0, 0]
    wiv = wiv_s[:]
    wmiv2 = wmiv_s[:]
    total = 0.0
    for g_idx in range(NSTREAM):
        xb = x_refs[g_idx][:]
        row = jnp.sum(xb * (xb * wiv - wmiv2), axis=1)
        elbo_refs[g_idx][:] = (-0.5 * (row + c)).reshape(SUB, 1)
        total += jnp.sum(row)
    acc_s[0, 0] += total

    @pl.when(i == GRID - 1)
    def _epilogue():
        loss_ref[:] = jnp.full((1, 1), 0.5 * (acc_s[0, 0] / B + c_s[0, 0]),
                               dtype=jnp.float32)


def kernel(x, categorical_weights, mus, log_var):
    key = jax.random.key(42)
    u = jax.random.uniform(key, (N_SAMPLES, K), jnp.float32,
                           minval=jnp.finfo(jnp.float32).tiny, maxval=1.0)
    cw = categorical_weights.reshape(1, K)

    x_specs = [
        pl.BlockSpec((SUB, D), functools.partial(lambda g, i: (g * GRID + i, 0), g))
        for g in range(NSTREAM)
    ]
    outs = pl.pallas_call(
        _mix_kernel,
        grid=(GRID,),
        in_specs=[
            pl.BlockSpec((N_SAMPLES, K), lambda i: (0, 0)),
            pl.BlockSpec((1, K), lambda i: (0, 0)),
            pl.BlockSpec((K, D), lambda i: (0, 0)),
            pl.BlockSpec((K, D), lambda i: (0, 0)),
            *x_specs,
        ],
        out_specs=[
            *[pl.BlockSpec((SUB, 1), lambda i: (i, 0)) for _ in range(NSTREAM)],
            pl.BlockSpec((1, 1), lambda i: (0, 0)),
        ],
        out_shape=[
            *[jax.ShapeDtypeStruct((GROUP, 1), jnp.float32) for _ in range(NSTREAM)],
            jax.ShapeDtypeStruct((1, 1), jnp.float32),
        ],
        scratch_shapes=[
            pltpu.VMEM((1, D), jnp.float32),
            pltpu.VMEM((1, D), jnp.float32),
            pltpu.SMEM((1, 1), jnp.float32),
            pltpu.SMEM((1, 1), jnp.float32),
        ],
    )(u, cw, mus, log_var, *([x] * NSTREAM))

    elbo = jnp.concatenate(outs[:NSTREAM], axis=0)[:, 0]
    return outs[NSTREAM][0, 0], elbo
